# baseline (speedup 1.0000x reference)
import functools

import jax
import jax.numpy as jnp
from jax import lax
from jax.experimental import pallas as pl
from jax.experimental.pallas import tpu as pltpu
from jax.experimental.pallas import tpu_sc as plsc

NUM_BINS = 100000

_INFO = plsc.get_sparse_core_info()
_NC = _INFO.num_cores
_NS = _INFO.num_subcores
_NL = _INFO.num_lanes
_NW = _NC * _NS

_TOTAL = 16384 * 26
_PER_W = _TOTAL // _NW
_NVEC = _PER_W // _NL

_mesh = plsc.VectorSubcoreMesh(core_axis_name="c", subcore_axis_name="s")


@functools.partial(
    pl.kernel,
    mesh=_mesh,
    out_type=jax.ShapeDtypeStruct((_TOTAL,), jnp.int32),
    scratch_types=[
        pltpu.VMEM((_PER_W,), jnp.int32),
        pltpu.VMEM((_PER_W,), jnp.int32),
    ],
)
def _hash_sc(x_hbm, out_hbm, x_v, o_v):
    wid = lax.axis_index("s") * _NC + lax.axis_index("c")
    base = wid * _PER_W
    pltpu.sync_copy(x_hbm.at[pl.ds(base, _PER_W)], x_v)

    def body(i, carry):
        off = pl.multiple_of(i * _NL, _NL)
        h = x_v[pl.ds(off, _NL)].astype(jnp.uint32)
        h = h ^ (h >> jnp.uint32(16))
        h = h * jnp.uint32(0x85EBCA6B)
        h = h ^ (h >> jnp.uint32(13))
        h = h * jnp.uint32(0xC2B2AE35)
        h = h ^ (h >> jnp.uint32(16))
        o_v[pl.ds(off, _NL)] = (h % jnp.uint32(NUM_BINS)).astype(jnp.int32)
        return carry

    lax.fori_loop(0, _NVEC, body, 0)
    pltpu.sync_copy(o_v, out_hbm.at[pl.ds(base, _PER_W)])


def kernel(inputs):
    flat = inputs.reshape(_TOTAL)
    return _hash_sc(flat).reshape(inputs.shape)


# speedup vs baseline: 1.0071x; 1.0071x over previous
import functools

import jax
import jax.numpy as jnp
from jax import lax
from jax.experimental import pallas as pl
from jax.experimental.pallas import tpu as pltpu
from jax.experimental.pallas import tpu_sc as plsc

NUM_BINS = 100000

_INFO = plsc.get_sparse_core_info()
_NC = _INFO.num_cores
_NS = _INFO.num_subcores
_NL = _INFO.num_lanes
_NW = _NC * _NS

_TOTAL = 16384 * 26
_PER_W = _TOTAL // _NW

_mesh = plsc.VectorSubcoreMesh(core_axis_name="c", subcore_axis_name="s")


@functools.partial(
    pl.kernel,
    mesh=_mesh,
    out_type=jax.ShapeDtypeStruct((_TOTAL,), jnp.int32),
    scratch_types=[pltpu.VMEM((_PER_W,), jnp.int32)],
)
def _hash_sc(x_hbm, out_hbm, x_v):
    wid = lax.axis_index("s") * _NC + lax.axis_index("c")
    base = wid * _PER_W
    pltpu.sync_copy(x_hbm.at[pl.ds(base, _PER_W)], x_v)

    @plsc.parallel_loop(0, _PER_W, step=_NL, unroll=8)
    def _(off):
        h = x_v[pl.ds(off, _NL)].astype(jnp.uint32)
        h = h ^ (h >> jnp.uint32(16))
        h = h * jnp.uint32(0x85EBCA6B)
        h = h ^ (h >> jnp.uint32(13))
        h = h * jnp.uint32(0xC2B2AE35)
        h = h ^ (h >> jnp.uint32(16))
        x_v[pl.ds(off, _NL)] = (h % jnp.uint32(NUM_BINS)).astype(jnp.int32)

    pltpu.sync_copy(x_v, out_hbm.at[pl.ds(base, _PER_W)])


def kernel(inputs):
    flat = inputs.reshape(_TOTAL)
    return _hash_sc(flat).reshape(inputs.shape)


# speedup vs baseline: 1.0143x; 1.0072x over previous
import functools

import jax
import jax.numpy as jnp
from jax import lax
from jax.experimental import pallas as pl
from jax.experimental.pallas import tpu as pltpu
from jax.experimental.pallas import tpu_sc as plsc

NUM_BINS = 100000

_INFO = plsc.get_sparse_core_info()
_NC = _INFO.num_cores
_NS = _INFO.num_subcores
_NL = _INFO.num_lanes
_NW = _NC * _NS

_TOTAL = 16384 * 26
_PER_W = _TOTAL // _NW
_NCH = 4
_CHUNK = _PER_W // _NCH

_mesh = plsc.VectorSubcoreMesh(core_axis_name="c", subcore_axis_name="s")


@functools.partial(
    pl.kernel,
    mesh=_mesh,
    out_type=jax.ShapeDtypeStruct((_TOTAL,), jnp.int32),
    scratch_types=[pltpu.VMEM((_PER_W,), jnp.int32)]
    + [pltpu.SemaphoreType.DMA] * (2 * _NCH),
)
def _hash_sc(x_hbm, out_hbm, x_v, *sems):
    in_sems, out_sems = sems[:_NCH], sems[_NCH:]
    wid = lax.axis_index("s") * _NC + lax.axis_index("c")
    base = wid * _PER_W

    in_handles = [
        pltpu.async_copy(
            x_hbm.at[pl.ds(base + c * _CHUNK, _CHUNK)],
            x_v.at[pl.ds(c * _CHUNK, _CHUNK)],
            in_sems[c],
        )
        for c in range(_NCH)
    ]
    out_handles = []
    for c in range(_NCH):
        in_handles[c].wait()

        @plsc.parallel_loop(c * _CHUNK, (c + 1) * _CHUNK, step=_NL, unroll=8)
        def _(off):
            h = x_v[pl.ds(off, _NL)].astype(jnp.uint32)
            h = h ^ (h >> jnp.uint32(16))
            h = h * jnp.uint32(0x85EBCA6B)
            h = h ^ (h >> jnp.uint32(13))
            h = h * jnp.uint32(0xC2B2AE35)
            h = h ^ (h >> jnp.uint32(16))
            x_v[pl.ds(off, _NL)] = (h % jnp.uint32(NUM_BINS)).astype(jnp.int32)

        out_handles.append(
            pltpu.async_copy(
                x_v.at[pl.ds(c * _CHUNK, _CHUNK)],
                out_hbm.at[pl.ds(base + c * _CHUNK, _CHUNK)],
                out_sems[c],
            )
        )
    for h in out_handles:
        h.wait()


def kernel(inputs):
    flat = inputs.reshape(_TOTAL)
    return _hash_sc(flat).reshape(inputs.shape)
